# baseline (speedup 1.0000x reference)
import functools

import jax
import jax.numpy as jnp
from jax import lax
from jax.experimental import pallas as pl
from jax.experimental.pallas import tpu as pltpu
from jax.experimental.pallas import tpu_sc as plsc

BATCH = 4096
SEQ_LEN = 200
EMBED_DIM = 64

NUM_CORES = 2
NUM_SUBCORES = 16
NUM_WORKERS = NUM_CORES * NUM_SUBCORES

SEQ_PER_WORKER = BATCH // NUM_WORKERS
CHUNK_SEQ = 4
CHUNK_ROWS = CHUNK_SEQ * SEQ_LEN
GATHER_SIZE = 100
GATHERS_PER_CHUNK = CHUNK_ROWS // GATHER_SIZE
NUM_CHUNKS = SEQ_PER_WORKER // CHUNK_SEQ
LANES = 16
GROUPS = EMBED_DIM // LANES


def _body(idx_hbm, table_hbm, pos_hbm, out_hbm, idx_v, rows_v, pos_v, sem):
    wid = lax.axis_index("s") * NUM_CORES + lax.axis_index("c")

    pltpu.sync_copy(pos_hbm, pos_v)

    def chunk_body(it, _):
        seq_base = wid * SEQ_PER_WORKER + it * CHUNK_SEQ
        idx_row_base = seq_base * (SEQ_LEN // GATHER_SIZE)
        pltpu.sync_copy(idx_hbm.at[pl.ds(idx_row_base, GATHERS_PER_CHUNK)],
                        idx_v)

        copies = []
        for j in range(GATHERS_PER_CHUNK):
            s, h = j // 2, j % 2
            dst = rows_v.at[s, pl.ds(h * GATHER_SIZE, GATHER_SIZE)]
            copies.append(
                pltpu.make_async_copy(table_hbm.at[idx_v.at[j]], dst, sem))
        for c in copies:
            c.start()
        for c in copies:
            c.wait()

        def pos_body(p, _):
            for g in range(GROUPS):
                pv = pos_v[p, pl.ds(g * LANES, LANES)]
                for s in range(CHUNK_SEQ):
                    rows_v[s, p, pl.ds(g * LANES, LANES)] += pv
            return _

        lax.fori_loop(0, SEQ_LEN, pos_body, 0, unroll=False)

        pltpu.sync_copy(rows_v, out_hbm.at[pl.ds(seq_base, CHUNK_SEQ)])
        return _

    lax.fori_loop(0, NUM_CHUNKS, chunk_body, 0, unroll=False)


@jax.jit
def _embed(inputs, token_table, pos_table):
    idx = inputs.reshape(BATCH * SEQ_LEN // GATHER_SIZE, GATHER_SIZE)
    mesh = plsc.VectorSubcoreMesh(
        core_axis_name="c", subcore_axis_name="s", num_cores=NUM_CORES,
        num_subcores=NUM_SUBCORES)
    f = pl.kernel(
        _body,
        out_type=jax.ShapeDtypeStruct((BATCH, SEQ_LEN, EMBED_DIM),
                                      jnp.float32),
        mesh=mesh,
        scratch_types=[
            pltpu.VMEM((GATHERS_PER_CHUNK, GATHER_SIZE), jnp.int32),
            pltpu.VMEM((CHUNK_SEQ, SEQ_LEN, EMBED_DIM), jnp.float32),
            pltpu.VMEM((SEQ_LEN, EMBED_DIM), jnp.float32),
            pltpu.SemaphoreType.DMA,
        ],
        compiler_params=pltpu.CompilerParams(use_tc_tiling_on_sc=False),
    )
    return f(idx, token_table, pos_table)


def kernel(inputs, token_table, pos_table):
    return _embed(inputs, token_table, pos_table)


# speedup vs baseline: 1.0624x; 1.0624x over previous
import jax
import jax.numpy as jnp
from jax import lax
from jax.experimental import pallas as pl
from jax.experimental.pallas import tpu as pltpu
from jax.experimental.pallas import tpu_sc as plsc

BATCH = 4096
SEQ_LEN = 200
EMBED_DIM = 64

NUM_CORES = 2
NUM_SUBCORES = 16
NUM_WORKERS = NUM_CORES * NUM_SUBCORES

SEQ_PER_WORKER = BATCH // NUM_WORKERS
CHUNK_SEQ = 2
CHUNK_ROWS = CHUNK_SEQ * SEQ_LEN
GATHER_SIZE = 100
GATHERS_PER_CHUNK = CHUNK_ROWS // GATHER_SIZE
NUM_CHUNKS = SEQ_PER_WORKER // CHUNK_SEQ
NBUF = 4
LANES = 16
GROUPS = EMBED_DIM // LANES
IDX_ROWS_PER_SEQ = SEQ_LEN // GATHER_SIZE


def _body(idx_hbm, table_hbm, pos_hbm, out_hbm, idx_v, rows_v, pos_v,
          gsems, wsems):
    wid = lax.axis_index("s") * NUM_CORES + lax.axis_index("c")
    seq0 = wid * SEQ_PER_WORKER

    pltpu.sync_copy(pos_hbm, pos_v)

    def gather_copies(b, c):
        idx_row_base = (seq0 + c * CHUNK_SEQ) * IDX_ROWS_PER_SEQ
        copies = []
        for j in range(GATHERS_PER_CHUNK):
            s, h = divmod(j, IDX_ROWS_PER_SEQ)
            dst = rows_v.at[b, s, pl.ds(h * GATHER_SIZE, GATHER_SIZE)]
            copies.append(
                pltpu.make_async_copy(table_hbm.at[idx_v.at[b, j]], dst,
                                      gsems[b]))
        return copies, idx_row_base

    def fire(b, c):
        copies, idx_row_base = gather_copies(b, c)
        pltpu.sync_copy(idx_hbm.at[pl.ds(idx_row_base, GATHERS_PER_CHUNK)],
                        idx_v.at[b])
        for cp in copies:
            cp.start()

    def write_copy(b, c):
        return pltpu.make_async_copy(
            rows_v.at[b], out_hbm.at[pl.ds(seq0 + c * CHUNK_SEQ, CHUNK_SEQ)],
            wsems[b])

    def pos_add(b):
        def pbody(p, carry):
            for g in range(GROUPS):
                pv = pos_v[p, pl.ds(g * LANES, LANES)]
                for s in range(CHUNK_SEQ):
                    rows_v[b, s, p, pl.ds(g * LANES, LANES)] += pv
            return carry

        lax.fori_loop(0, SEQ_LEN, pbody, 0, unroll=False)

    for b in range(NBUF):
        fire(b, b)

    def outer(o, carry):
        for b in range(NBUF):
            c = o * NBUF + b
            bp = (b - 1) % NBUF
            copies, _ = gather_copies(b, c)
            for cp in copies:
                cp.wait()
            pos_add(b)

            @pl.when(c >= 1)
            def _():
                write_copy(bp, c - 1).wait()

            @pl.when(jnp.logical_and(c >= 1, c <= NUM_CHUNKS - NBUF))
            def _():
                fire(bp, c + NBUF - 1)

            write_copy(b, c).start()
        return carry

    lax.fori_loop(0, NUM_CHUNKS // NBUF, outer, 0, unroll=False)

    write_copy((NUM_CHUNKS - 1) % NBUF, NUM_CHUNKS - 1).wait()


@jax.jit
def _embed(inputs, token_table, pos_table):
    idx = inputs.reshape(BATCH * SEQ_LEN // GATHER_SIZE, GATHER_SIZE)
    mesh = plsc.VectorSubcoreMesh(
        core_axis_name="c", subcore_axis_name="s", num_cores=NUM_CORES,
        num_subcores=NUM_SUBCORES)
    f = pl.kernel(
        _body,
        out_type=jax.ShapeDtypeStruct((BATCH, SEQ_LEN, EMBED_DIM),
                                      jnp.float32),
        mesh=mesh,
        scratch_types=[
            pltpu.VMEM((NBUF, GATHERS_PER_CHUNK, GATHER_SIZE), jnp.int32),
            pltpu.VMEM((NBUF, CHUNK_SEQ, SEQ_LEN, EMBED_DIM), jnp.float32),
            pltpu.VMEM((SEQ_LEN, EMBED_DIM), jnp.float32),
            [pltpu.SemaphoreType.DMA] * NBUF,
            [pltpu.SemaphoreType.DMA] * NBUF,
        ],
        compiler_params=pltpu.CompilerParams(use_tc_tiling_on_sc=False),
    )
    return f(idx, token_table, pos_table)


def kernel(inputs, token_table, pos_table):
    return _embed(inputs, token_table, pos_table)
